# baseline (speedup 1.0000x reference)
import jax
import jax.numpy as jnp
from jax import lax
from jax.experimental import pallas as pl
from jax.experimental.pallas import tpu as pltpu
from jax.experimental.pallas import tpu_sc as plsc

B = 4
IN_SZ = 40962
OUT_SZ = 163842
D = 128
W = B * D
NEW = OUT_SZ - IN_SZ
NC, NS = 2, 16
NW = NC * NS

GPW = NEW // NW
K = 48
T = GPW // K

IPW = 1280
IK = 16
IC = IPW // IK
ITAIL = IN_SZ - NW * IPW

_mesh = plsc.VectorSubcoreMesh(
    core_axis_name="c", subcore_axis_name="s", num_cores=NC, num_subcores=NS)
_params = pltpu.CompilerParams(use_tc_tiling_on_sc=False)


def _upsample_body(xv_hbm, li_hbm, ri_hbm, out_hbm,
                   idx_l, idx_r, rl0, rr0, rl1, rr1, id0, id1,
                   s_g0, s_g1, s_o0, s_o1, s_ii0, s_ii1, s_io0, s_io1):
    wid = lax.axis_index("s") * NC + lax.axis_index("c")
    rls = (rl0, rl1)
    rrs = (rr0, rr1)
    ids = (id0, id1)
    gsems = (s_g0, s_g1)
    osems = (s_o0, s_o1)
    iisems = (s_ii0, s_ii1)
    iosems = (s_io0, s_io1)

    cl = pltpu.async_copy(li_hbm.at[pl.ds(wid * GPW, GPW)], idx_l, s_g0)
    cr = pltpu.async_copy(ri_hbm.at[pl.ds(wid * GPW, GPW)], idx_r, s_g1)
    cl.wait()
    cr.wait()

    def issue_gather(t, p):
        off = t * K
        pltpu.async_copy(xv_hbm.at[idx_l.at[pl.ds(off, K)]], rls[p], gsems[p])
        pltpu.async_copy(xv_hbm.at[idx_r.at[pl.ds(off, K)]], rrs[p], gsems[p])

    def wait_gather(p):
        pltpu.make_async_copy(xv_hbm.at[pl.ds(0, K)], rls[p], gsems[p]).wait()
        pltpu.make_async_copy(xv_hbm.at[pl.ds(0, K)], rrs[p], gsems[p]).wait()

    def issue_out(t, p):
        pltpu.async_copy(rls[p],
                         out_hbm.at[pl.ds(IN_SZ + wid * GPW + t * K, K)],
                         osems[p])

    def wait_out(p):
        pltpu.make_async_copy(xv_hbm.at[pl.ds(0, K)], rls[p], osems[p]).wait()

    def avg(p):
        rl, rr = rls[p], rrs[p]

        def row(i, carry):
            for j in range(W // 16):
                s = pl.ds(j * 16, 16)
                rl[i, s] = (rl[i, s] + rr[i, s]) * 0.5
            return carry

        lax.fori_loop(0, K, row, 0)

    def issue_id_in(c, p):
        pltpu.async_copy(xv_hbm.at[pl.ds(wid * IPW + c * IK, IK)], ids[p],
                         iisems[p])

    def wait_id_in(p):
        pltpu.make_async_copy(xv_hbm.at[pl.ds(0, IK)], ids[p],
                              iisems[p]).wait()

    def issue_id_out(c, p):
        pltpu.async_copy(ids[p], out_hbm.at[pl.ds(wid * IPW + c * IK, IK)],
                         iosems[p])

    def wait_id_out(p):
        pltpu.make_async_copy(xv_hbm.at[pl.ds(0, IK)], ids[p],
                              iosems[p]).wait()

    def step(t, p, first=False, last=None):
        q = 1 - p
        if not first:
            wait_out(q)
            wait_id_out(q)
        if last is None:
            issue_gather(t + 1, q)
            issue_id_in(t + 1, q)
        else:
            def _issue_next():
                issue_gather(t + 1, q)
                issue_id_in(t + 1, q)

            pl.when(jnp.logical_not(last))(_issue_next)
        wait_gather(p)
        avg(p)
        issue_out(t, p)
        wait_id_in(p)
        issue_id_out(t, p)

    issue_gather(0, 0)
    issue_id_in(0, 0)
    step(0, 0, first=True)
    step(1, 1)

    def two_steps(k, carry):
        t0 = 2 * k
        step(t0, 0)
        step(t0 + 1, 1, last=(k == T // 2 - 1))
        return carry

    lax.fori_loop(1, T // 2, two_steps, 0)

    wait_out(1)
    wait_id_out(1)

    @pl.when(wid == 0)
    def _tails():
        r0 = NW * IPW
        pltpu.sync_copy(xv_hbm.at[pl.ds(r0, ITAIL)], id0.at[pl.ds(0, ITAIL)])
        pltpu.sync_copy(id0.at[pl.ds(0, ITAIL)],
                        out_hbm.at[pl.ds(r0, ITAIL)])

_upsample = pl.kernel(
    _upsample_body,
    out_type=jax.ShapeDtypeStruct((OUT_SZ, W), jnp.float32),
    mesh=_mesh,
    compiler_params=_params,
    scratch_types=[
        pltpu.VMEM((GPW,), jnp.int32),
        pltpu.VMEM((GPW,), jnp.int32),
        pltpu.VMEM((K, W), jnp.float32),
        pltpu.VMEM((K, W), jnp.float32),
        pltpu.VMEM((K, W), jnp.float32),
        pltpu.VMEM((K, W), jnp.float32),
        pltpu.VMEM((IK, W), jnp.float32),
        pltpu.VMEM((IK, W), jnp.float32),
        pltpu.SemaphoreType.DMA,
        pltpu.SemaphoreType.DMA,
        pltpu.SemaphoreType.DMA,
        pltpu.SemaphoreType.DMA,
        pltpu.SemaphoreType.DMA,
        pltpu.SemaphoreType.DMA,
        pltpu.SemaphoreType.DMA,
        pltpu.SemaphoreType.DMA,
    ],
)


def kernel(x, left_idx, right_idx):
    xv = jnp.transpose(x, (1, 0, 2)).reshape(IN_SZ, W)
    li = left_idx[IN_SZ:].astype(jnp.int32)
    ri = right_idx[IN_SZ:].astype(jnp.int32)
    outv = _upsample(xv, li, ri)
    return jnp.transpose(outv.reshape(OUT_SZ, B, D), (1, 0, 2))
